# baseline (speedup 1.0000x reference)
import jax
import jax.numpy as jnp
from jax import lax
from jax.experimental import pallas as pl

_B = 32
_A = 102
_S = 224 * 224
_ROWS = 8


def _pool_body(x_ref, o_ref):
    o_ref[...] = jnp.sum(x_ref[...], axis=1, keepdims=True) * (1.0 / _S)


def _gate_body(x_ref, w_ref, o_ref):
    x = x_ref[...]
    w = w_ref[...]
    vote = x * w
    absv = jnp.abs(vote)
    dense = jnp.sum(vote, axis=1)
    iota = lax.broadcasted_iota(jnp.int32, (_B, _A), 1)
    acc = jnp.zeros((_B,), jnp.float32)
    outs = []
    for _ in range(8):
        m = jnp.max(absv, axis=1, keepdims=True)
        ismax = absv == m
        first = jnp.min(jnp.where(ismax, iota, _A), axis=1, keepdims=True)
        onehot = iota == first
        acc = acc + jnp.sum(jnp.where(onehot, vote, 0.0), axis=1)
        outs.append(acc + 0.5)
        absv = jnp.where(onehot, -1.0, absv)
    outs.append(dense + 0.5)
    o_ref[...] = jnp.stack(outs, axis=0)


def kernel(maps, W2):
    maps2 = maps.reshape(_B * _A, _S)
    sums = pl.pallas_call(
        _pool_body,
        grid=(_B * _A // _ROWS,),
        in_specs=[pl.BlockSpec((_ROWS, _S), lambda i: (i, 0))],
        out_specs=pl.BlockSpec((_ROWS, 1), lambda i: (i, 0)),
        out_shape=jax.ShapeDtypeStruct((_B * _A, 1), jnp.float32),
    )(maps2)
    x_sun = sums.reshape(_B, _A)

    son = pl.pallas_call(
        _gate_body,
        out_shape=jax.ShapeDtypeStruct((9, _B), jnp.float32),
    )(x_sun, W2)
    x_son = son.reshape(9, _B, 1)
    return (x_sun, x_son, maps)


# speedup vs baseline: 1.0114x; 1.0114x over previous
import jax
import jax.numpy as jnp
from jax import lax
from jax.experimental import pallas as pl

_B = 32
_A = 102
_S = 224 * 224
_LANES = 128
_GRPS = _S // _LANES
_ROWS = 32


def _pool1_body(x_ref, o_ref):
    o_ref[...] = jnp.sum(x_ref[...], axis=1)


def _pool2_body(p_ref, o_ref):
    o_ref[...] = jnp.sum(p_ref[...], axis=1, keepdims=True) * (1.0 / _S)


def _gate_body(x_ref, w_ref, o_ref):
    x = x_ref[...]
    w = w_ref[...]
    vote = x * w
    absv = jnp.abs(vote)
    dense = jnp.sum(vote, axis=1)
    iota = lax.broadcasted_iota(jnp.int32, (_B, _A), 1)
    acc = jnp.zeros((_B,), jnp.float32)
    outs = []
    for _ in range(8):
        m = jnp.max(absv, axis=1, keepdims=True)
        ismax = absv == m
        first = jnp.min(jnp.where(ismax, iota, _A), axis=1, keepdims=True)
        onehot = iota == first
        acc = acc + jnp.sum(jnp.where(onehot, vote, 0.0), axis=1)
        outs.append(acc + 0.5)
        absv = jnp.where(onehot, -1.0, absv)
    outs.append(dense + 0.5)
    o_ref[...] = jnp.stack(outs, axis=0)


def kernel(maps, W2):
    n = _B * _A
    maps3 = maps.reshape(n, _GRPS, _LANES)
    partials = pl.pallas_call(
        _pool1_body,
        grid=(n // _ROWS,),
        in_specs=[pl.BlockSpec((_ROWS, _GRPS, _LANES), lambda i: (i, 0, 0))],
        out_specs=pl.BlockSpec((_ROWS, _LANES), lambda i: (i, 0)),
        out_shape=jax.ShapeDtypeStruct((n, _LANES), jnp.float32),
    )(maps3)
    sums = pl.pallas_call(
        _pool2_body,
        out_shape=jax.ShapeDtypeStruct((n, 1), jnp.float32),
    )(partials)
    x_sun = sums.reshape(_B, _A)

    son = pl.pallas_call(
        _gate_body,
        out_shape=jax.ShapeDtypeStruct((9, _B), jnp.float32),
    )(x_sun, W2)
    x_son = son.reshape(9, _B, 1)
    return (x_sun, x_son, maps)
